# baseline (speedup 1.0000x reference)
import functools

import jax
import jax.numpy as jnp
from jax import lax
from jax.experimental import pallas as pl
from jax.experimental.pallas import tpu as pltpu
from jax.experimental.pallas import tpu_sc as plsc

N = 50000
D = 128
H = 64
G = 32
_BLK = 2000


def _stats(h, a_s, a_d):
    hs = jnp.dot(h, a_s, preferred_element_type=jnp.float32)
    hd = jnp.dot(h, a_d, preferred_element_type=jnp.float32)
    return hs, hd


def _mm1_body(x_ref, w_ref, as_ref, ad_ref, h_ref, hs_ref, hd_ref):
    h = jnp.dot(x_ref[...], w_ref[...], preferred_element_type=jnp.float32)
    hs, hd = _stats(h, as_ref[...], ad_ref[...])
    h_ref[...] = h
    hs_ref[...] = hs
    hd_ref[...] = hd


def _ln(y0, g, bb):
    mu = jnp.mean(y0, axis=-1, keepdims=True)
    v = jnp.mean((y0 - mu) ** 2, axis=-1, keepdims=True)
    return (y0 - mu) * jax.lax.rsqrt(v + 1e-5) * g + bb


def _lnmm_body(acc_ref, s_ref, b_ref, g_ref, bb_ref, w_ref, as_ref, ad_ref,
               h_ref, hs_ref, hd_ref):
    y0 = acc_ref[...] / (s_ref[...] + 1e-16) + b_ref[...]
    y = jnp.maximum(_ln(y0, g_ref[...], bb_ref[...]), 0.0)
    h = jnp.dot(y, w_ref[...], preferred_element_type=jnp.float32)
    hs, hd = _stats(h, as_ref[...], ad_ref[...])
    h_ref[...] = h
    hs_ref[...] = hs
    hd_ref[...] = hd


def _lnpool_body(acc_ref, s_ref, b_ref, g_ref, bb_ref, batch_ref,
                 emb_ref, scr_ref):
    i = pl.program_id(0)

    @pl.when(i == 0)
    def _():
        scr_ref[...] = jnp.zeros_like(scr_ref)

    y0 = acc_ref[...] / (s_ref[...] + 1e-16) + b_ref[...]
    y = jnp.maximum(_ln(y0, g_ref[...], bb_ref[...]), 0.0)
    onehot = (batch_ref[...] == jax.lax.broadcasted_iota(jnp.int32, (1, G), 1)
              ).astype(jnp.float32)
    y_aug = jnp.concatenate([y, jnp.ones_like(y[:, :1])], axis=1)
    scr_ref[...] += jax.lax.dot_general(
        onehot, y_aug, (((0,), (0,)), ((), ())),
        preferred_element_type=jnp.float32)

    @pl.when(i == pl.num_programs(0) - 1)
    def _():
        sums = scr_ref[:, :H]
        cnts = jnp.clip(scr_ref[:, H:H + 1], 1.0, None)
        emb_ref[...] = sums / cnts


def _row_spec(width):
    return pl.BlockSpec((_BLK, width), lambda i: (i, 0))


def _full_spec(shape):
    return pl.BlockSpec(shape, lambda i: tuple(0 for _ in shape))


def _mm1(x, W, a_s, a_d):
    grid = (N // _BLK,)
    outs = (
        jax.ShapeDtypeStruct((N, H), jnp.float32),
        jax.ShapeDtypeStruct((N, 1), jnp.float32),
        jax.ShapeDtypeStruct((N, 1), jnp.float32),
    )
    return pl.pallas_call(
        _mm1_body,
        grid=grid,
        in_specs=[_row_spec(D), _full_spec((D, H)), _full_spec((H, 1)),
                  _full_spec((H, 1))],
        out_specs=[_row_spec(H), _row_spec(1), _row_spec(1)],
        out_shape=outs,
    )(x, W, a_s.reshape(H, 1), a_d.reshape(H, 1))


def _lnmm(acc, s, b, g, bb, W, a_s, a_d):
    grid = (N // _BLK,)
    outs = (
        jax.ShapeDtypeStruct((N, H), jnp.float32),
        jax.ShapeDtypeStruct((N, 1), jnp.float32),
        jax.ShapeDtypeStruct((N, 1), jnp.float32),
    )
    return pl.pallas_call(
        _lnmm_body,
        grid=grid,
        in_specs=[_row_spec(H), _row_spec(1), _full_spec((1, H)),
                  _full_spec((1, H)), _full_spec((1, H)), _full_spec((H, H)),
                  _full_spec((H, 1)), _full_spec((H, 1))],
        out_specs=[_row_spec(H), _row_spec(1), _row_spec(1)],
        out_shape=outs,
    )(acc, s, b.reshape(1, H), g.reshape(1, H), bb.reshape(1, H), W,
      a_s.reshape(H, 1), a_d.reshape(H, 1))


def _lnpool(acc, s, b, g, bb, batch):
    grid = (N // _BLK,)
    return pl.pallas_call(
        _lnpool_body,
        grid=grid,
        in_specs=[_row_spec(H), _row_spec(1), _full_spec((1, H)),
                  _full_spec((1, H)), _full_spec((1, H)), _row_spec(1)],
        out_specs=pl.BlockSpec((G, H), lambda i: (0, 0)),
        out_shape=jax.ShapeDtypeStruct((G, H), jnp.float32),
        scratch_shapes=[pltpu.VMEM((G, H + 1), jnp.float32)],
    )(acc, s, b.reshape(1, H), g.reshape(1, H), bb.reshape(1, H),
      batch.reshape(N, 1).astype(jnp.int32))


E = 800000
_HALF = N // 2
_STRIP = 1568
_NP = 16 * _STRIP
_K = 128
_NECHUNK = E // _K
_NSCHUNK = N // _K
_SELF_TAIL = N - _NSCHUNK * _K
_NCHUNK = _NECHUNK + _NSCHUNK
_ROUNDS = _NCHUNK // 16
_EP = E + N + 16
_SSTRIP = _NP // 16


def _splat(v16, e):
    return lax.gather(
        v16, jnp.full((16, 1), e, jnp.int32),
        dimension_numbers=lax.GatherDimensionNumbers(
            offset_dims=(), collapsed_slice_dims=(0,), start_index_map=(0,)),
        slice_sizes=(1,),
        mode=lax.GatherScatterMode.PROMISE_IN_BOUNDS)


def _attn_body(hs_hbm, hd_hbm, src_hbm, dst_hbm,
               p_out, s_out,
               hs_v, hd_v, src_v, dst_v, pidx_v, dloc_v, p_v, s_sh):
    c = lax.axis_index("c")
    t = lax.axis_index("s")
    base = pl.multiple_of(c * _HALF, 8)
    pbase = pl.multiple_of(c * _EP, 8)
    dummy = _HALF + t
    off = pl.multiple_of(t * _SSTRIP, 32)

    z16f = jnp.zeros((16,), jnp.float32)
    for j in range(_K // 16):
        p_v[pl.ds(j * 16, 16)] = z16f
    for k in range(_SSTRIP // _K):
        pltpu.sync_copy(p_v, s_sh.at[pl.ds(off + k * _K, _K)])
    pltpu.sync_copy(p_v.at[pl.ds(0, _SSTRIP % _K)],
                    s_sh.at[pl.ds(off + (_SSTRIP // _K) * _K, _SSTRIP % _K)])

    pltpu.sync_copy(hs_hbm, hs_v)
    pltpu.sync_copy(hd_hbm.at[pl.ds(base, _HALF)], hd_v.at[pl.ds(0, _HALF)])
    plsc.subcore_barrier()

    def group(j, s16, d16, gidx16):
        sl = pl.ds(j * 16, 16)
        hsg = plsc.load_gather(hs_v, [s16])
        dl16 = jnp.clip(d16 - base, 0, _HALF - 1)
        hdg = plsc.load_gather(hd_v, [dl16])
        l = hsg + hdg
        p16 = jnp.exp(jnp.where(l >= 0, l, 0.2 * l))
        owned = (d16 >= base) & (d16 < base + _HALF)
        dloc_v[sl] = jnp.where(owned, dl16, dummy)
        pidx_v[sl] = jnp.where(owned, gidx16,
                               _EP - 16 + lax.iota(jnp.int32, 16))
        p_v[sl] = p16

    def pad_tail(ngroups):
        dummy16 = jnp.full((16,), dummy, jnp.int32)
        pdummy16 = _EP - 16 + lax.iota(jnp.int32, 16)
        for j in range(ngroups, _K // 16):
            dloc_v[pl.ds(j * 16, 16)] = dummy16
            pidx_v[pl.ds(j * 16, 16)] = pdummy16

    def scatter_chunk():
        pltpu.sync_copy(p_v, p_out.at[pidx_v])
        pltpu.sync_copy(p_v, s_sh.at[dloc_v], add=True)

    def edge_chunk(cc):
        eoff = pl.multiple_of(cc * _K, _K)
        pltpu.sync_copy(src_hbm.at[pl.ds(eoff, _K)], src_v)
        pltpu.sync_copy(dst_hbm.at[pl.ds(eoff, _K)], dst_v)

        def grp(j, carry):
            sl = pl.ds(j * 16, 16)
            gidx16 = pbase + eoff + j * 16 + lax.iota(jnp.int32, 16)
            group(j, src_v[sl], dst_v[sl], gidx16)
            return carry

        lax.fori_loop(0, _K // 16, grp, 0)
        scatter_chunk()

    def self_chunk(cs, sz):
        noff = pl.multiple_of(cs * _K, _K)

        def grp(j, carry):
            n16 = noff + j * 16 + lax.iota(jnp.int32, 16)
            group(j, n16, n16, pbase + E + n16)
            return carry

        lax.fori_loop(0, sz // 16, grp, 0)
        if sz < _K:
            pad_tail(sz // 16)
        scatter_chunk()

    def round_(k, carry):
        cc = k * 16 + t
        lax.cond(cc < _NECHUNK,
                 lambda: edge_chunk(cc),
                 lambda: self_chunk(cc - _NECHUNK, _K))
        return carry

    lax.fori_loop(0, _ROUNDS, round_, 0)

    @pl.when(t == 0)
    def _():
        self_chunk(_NSCHUNK, _SELF_TAIL)

    plsc.subcore_barrier()
    for k in range(_SSTRIP // _K):
        pltpu.sync_copy(s_sh.at[pl.ds(off + k * _K, _K)], p_v)
        pltpu.sync_copy(p_v, s_out.at[pl.ds(c * _NP + off + k * _K, _K)])
    pltpu.sync_copy(s_sh.at[pl.ds(off + (_SSTRIP // _K) * _K, _SSTRIP % _K)],
                    p_v.at[pl.ds(0, _SSTRIP % _K)])
    pltpu.sync_copy(p_v.at[pl.ds(0, _SSTRIP % _K)],
                    s_out.at[pl.ds(c * _NP + off + (_SSTRIP // _K) * _K,
                                   _SSTRIP % _K)])


_attn_sc = functools.partial(
    pl.kernel,
    out_type=[jax.ShapeDtypeStruct((2 * _EP,), jnp.float32),
              jax.ShapeDtypeStruct((2 * _NP,), jnp.float32)],
    mesh=plsc.VectorSubcoreMesh(core_axis_name="c", subcore_axis_name="s"),
    compiler_params=pltpu.CompilerParams(needs_layout_passes=False,
                                         use_tc_tiling_on_sc=False),
    scratch_types=[
        pltpu.VMEM((N,), jnp.float32),
        pltpu.VMEM((_HALF + 24,), jnp.float32),
        pltpu.VMEM((_K,), jnp.int32),
        pltpu.VMEM((_K,), jnp.int32),
        pltpu.VMEM((_K,), jnp.int32),
        pltpu.VMEM((_K,), jnp.int32),
        pltpu.VMEM((_K,), jnp.float32),
        pltpu.VMEM_SHARED((_NP,), jnp.float32),
    ],
)(_attn_body)


def _rows_body(h_hbm, src_hbm, dst_hbm, p_hbm,
               acc_out,
               src_v, dst_v, dloc_v, p_v, rows_v, sem, acc_sh):
    c = lax.axis_index("c")
    t = lax.axis_index("s")
    base = pl.multiple_of(c * _HALF, 8)
    pbase = pl.multiple_of(c * _EP, 8)
    dummy = _HALF + t
    off = pl.multiple_of(t * _STRIP, 32)

    z16f = jnp.zeros((16,), jnp.float32)

    def zrow(r, carry):
        for q in range(H // 16):
            rows_v[r, pl.ds(q * 16, 16)] = z16f
        return carry

    lax.fori_loop(0, _K, zrow, 0)
    for k in range(12):
        pltpu.sync_copy(rows_v, acc_sh.at[pl.ds(off + k * _K, _K)])
    pltpu.sync_copy(rows_v.at[pl.ds(0, 32)],
                    acc_sh.at[pl.ds(off + 12 * _K, 32)])
    plsc.subcore_barrier()

    def scale(j):
        sl = pl.ds(j * 16, 16)
        d16 = dst_v[sl]
        owned = (d16 >= base) & (d16 < base + _HALF)
        dloc_v[sl] = jnp.where(owned, d16 - base, dummy)
        p16 = p_v[sl]
        for e in range(16):
            pe = _splat(p16, e)
            row = j * 16 + e
            for q in range(H // 16):
                cs = pl.ds(q * 16, 16)
                rows_v[row, cs] = rows_v[row, cs] * pe

    def edge_chunk(cc):
        eoff = pl.multiple_of(cc * _K, _K)
        pltpu.sync_copy(src_hbm.at[pl.ds(eoff, _K)], src_v)
        pltpu.sync_copy(dst_hbm.at[pl.ds(eoff, _K)], dst_v)
        pltpu.sync_copy(p_hbm.at[pl.ds(pbase + eoff, _K)], p_v)
        pltpu.async_copy(h_hbm.at[src_v], rows_v, sem).wait()

        def grp(j, carry):
            scale(j)
            return carry

        lax.fori_loop(0, _K // 16, grp, 0)
        pltpu.sync_copy(rows_v, acc_sh.at[dloc_v], add=True)

    def self_chunk(cs, sz):
        noff = pl.multiple_of(cs * _K, _K)
        pltpu.sync_copy(h_hbm.at[pl.ds(noff, sz)], rows_v.at[pl.ds(0, sz)])
        pltpu.sync_copy(p_hbm.at[pl.ds(pbase + E + noff, sz)],
                        p_v.at[pl.ds(0, sz)])

        def grp(j, carry):
            sl = pl.ds(j * 16, 16)
            dst_v[sl] = noff + j * 16 + lax.iota(jnp.int32, 16)
            scale(j)
            return carry

        lax.fori_loop(0, sz // 16, grp, 0)
        if sz < _K:
            dummy16 = jnp.full((16,), dummy, jnp.int32)
            for j in range(sz // 16, _K // 16):
                dloc_v[pl.ds(j * 16, 16)] = dummy16
        pltpu.sync_copy(rows_v, acc_sh.at[dloc_v], add=True)

    def round_(k, carry):
        cc = k * 16 + t
        lax.cond(cc < _NECHUNK,
                 lambda: edge_chunk(cc),
                 lambda: self_chunk(cc - _NECHUNK, _K))
        return carry

    lax.fori_loop(0, _ROUNDS, round_, 0)

    @pl.when(t == 0)
    def _():
        self_chunk(_NSCHUNK, _SELF_TAIL)

    plsc.subcore_barrier()
    for k in range(12):
        pltpu.sync_copy(acc_sh.at[pl.ds(off + k * _K, _K)], rows_v)
        pltpu.sync_copy(rows_v, acc_out.at[c, pl.ds(off + k * _K, _K)])
    pltpu.sync_copy(acc_sh.at[pl.ds(off + 12 * _K, 32)],
                    rows_v.at[pl.ds(0, 32)])
    pltpu.sync_copy(rows_v.at[pl.ds(0, 32)],
                    acc_out.at[c, pl.ds(off + 12 * _K, 32)])


_rows_sc = functools.partial(
    pl.kernel,
    out_type=[jax.ShapeDtypeStruct((2, _NP, H), jnp.float32)],
    mesh=plsc.VectorSubcoreMesh(core_axis_name="c", subcore_axis_name="s"),
    compiler_params=pltpu.CompilerParams(needs_layout_passes=False,
                                         use_tc_tiling_on_sc=False),
    scratch_types=[
        pltpu.VMEM((_K,), jnp.int32),
        pltpu.VMEM((_K,), jnp.int32),
        pltpu.VMEM((_K,), jnp.int32),
        pltpu.VMEM((_K,), jnp.float32),
        pltpu.VMEM((_K, H), jnp.float32),
        pltpu.SemaphoreType.DMA,
        pltpu.VMEM_SHARED((_NP, H), jnp.float32),
    ],
)(_rows_body)


def _edge_pass(h, hs, hd, src, dst):
    p_all, s_p = _attn_sc(hs.reshape(N), hd.reshape(N), src, dst)
    acc_p, = _rows_sc(h, src, dst, p_all)
    acc = jnp.concatenate([acc_p[0, :_HALF], acc_p[1, :_HALF]])
    s = jnp.concatenate([s_p[:_HALF], s_p[_NP:_NP + _HALF]])
    return acc, s.reshape(N, 1)


def kernel(x, edge_index, edge_attr, global_features, batch,
           W1, as1, ad1, b1, g1, bb1,
           W2, as2, ad2, b2, g2, bb2,
           W3, as3, ad3, b3, g3, bb3):
    src = edge_index[0]
    dst = edge_index[1]

    h, hs, hd = _mm1(x, W1, as1, ad1)
    acc, s = _edge_pass(h, hs, hd, src, dst)

    h, hs, hd = _lnmm(acc, s, b1, g1, bb1, W2, as2, ad2)
    acc, s = _edge_pass(h, hs, hd, src, dst)

    h, hs, hd = _lnmm(acc, s, b2, g2, bb2, W3, as3, ad3)
    acc, s = _edge_pass(h, hs, hd, src, dst)

    return _lnpool(acc, s, b3, g3, bb3, batch)


# speedup vs baseline: 49.6693x; 49.6693x over previous
import functools

import jax
import jax.numpy as jnp
from jax import lax
from jax.experimental import pallas as pl
from jax.experimental.pallas import tpu as pltpu
from jax.experimental.pallas import tpu_sc as plsc

N = 50000
D = 128
H = 64
G = 32
_BLK = 2000


def _stats(h, a_s, a_d):
    hs = jnp.dot(h, a_s, preferred_element_type=jnp.float32)
    hd = jnp.dot(h, a_d, preferred_element_type=jnp.float32)
    return hs, hd


def _mm1_body(x_ref, w_ref, as_ref, ad_ref, h_ref, hs_ref, hd_ref):
    h = jnp.dot(x_ref[...], w_ref[...], preferred_element_type=jnp.float32)
    hs, hd = _stats(h, as_ref[...], ad_ref[...])
    h_ref[...] = h
    hs_ref[...] = hs
    hd_ref[...] = hd


def _ln(y0, g, bb):
    mu = jnp.mean(y0, axis=-1, keepdims=True)
    v = jnp.mean((y0 - mu) ** 2, axis=-1, keepdims=True)
    return (y0 - mu) * jax.lax.rsqrt(v + 1e-5) * g + bb


def _lnmm_body(acc_ref, s_ref, b_ref, g_ref, bb_ref, w_ref, as_ref, ad_ref,
               h_ref, hs_ref, hd_ref):
    y0 = acc_ref[...] / (s_ref[...] + 1e-16) + b_ref[...]
    y = jnp.maximum(_ln(y0, g_ref[...], bb_ref[...]), 0.0)
    h = jnp.dot(y, w_ref[...], preferred_element_type=jnp.float32)
    hs, hd = _stats(h, as_ref[...], ad_ref[...])
    h_ref[...] = h
    hs_ref[...] = hs
    hd_ref[...] = hd


def _lnpool_body(acc_ref, s_ref, b_ref, g_ref, bb_ref, batch_ref,
                 emb_ref, scr_ref):
    i = pl.program_id(0)

    @pl.when(i == 0)
    def _():
        scr_ref[...] = jnp.zeros_like(scr_ref)

    y0 = acc_ref[...] / (s_ref[...] + 1e-16) + b_ref[...]
    y = jnp.maximum(_ln(y0, g_ref[...], bb_ref[...]), 0.0)
    onehot = (batch_ref[...] == jax.lax.broadcasted_iota(jnp.int32, (1, G), 1)
              ).astype(jnp.float32)
    y_aug = jnp.concatenate([y, jnp.ones_like(y[:, :1])], axis=1)
    scr_ref[...] += jax.lax.dot_general(
        onehot, y_aug, (((0,), (0,)), ((), ())),
        preferred_element_type=jnp.float32)

    @pl.when(i == pl.num_programs(0) - 1)
    def _():
        sums = scr_ref[:, :H]
        cnts = jnp.clip(scr_ref[:, H:H + 1], 1.0, None)
        emb_ref[...] = sums / cnts


def _row_spec(width):
    return pl.BlockSpec((_BLK, width), lambda i: (i, 0))


def _full_spec(shape):
    return pl.BlockSpec(shape, lambda i: tuple(0 for _ in shape))


def _mm1(x, W, a_s, a_d):
    grid = (N // _BLK,)
    outs = (
        jax.ShapeDtypeStruct((N, H), jnp.float32),
        jax.ShapeDtypeStruct((N, 1), jnp.float32),
        jax.ShapeDtypeStruct((N, 1), jnp.float32),
    )
    return pl.pallas_call(
        _mm1_body,
        grid=grid,
        in_specs=[_row_spec(D), _full_spec((D, H)), _full_spec((H, 1)),
                  _full_spec((H, 1))],
        out_specs=[_row_spec(H), _row_spec(1), _row_spec(1)],
        out_shape=outs,
    )(x, W, a_s.reshape(H, 1), a_d.reshape(H, 1))


def _lnmm(acc, s, b, g, bb, W, a_s, a_d):
    grid = (N // _BLK,)
    outs = (
        jax.ShapeDtypeStruct((N, H), jnp.float32),
        jax.ShapeDtypeStruct((N, 1), jnp.float32),
        jax.ShapeDtypeStruct((N, 1), jnp.float32),
    )
    return pl.pallas_call(
        _lnmm_body,
        grid=grid,
        in_specs=[_row_spec(H), _row_spec(1), _full_spec((1, H)),
                  _full_spec((1, H)), _full_spec((1, H)), _full_spec((H, H)),
                  _full_spec((H, 1)), _full_spec((H, 1))],
        out_specs=[_row_spec(H), _row_spec(1), _row_spec(1)],
        out_shape=outs,
    )(acc, s, b.reshape(1, H), g.reshape(1, H), bb.reshape(1, H), W,
      a_s.reshape(H, 1), a_d.reshape(H, 1))


def _lnpool(acc, s, b, g, bb, batch):
    grid = (N // _BLK,)
    return pl.pallas_call(
        _lnpool_body,
        grid=grid,
        in_specs=[_row_spec(H), _row_spec(1), _full_spec((1, H)),
                  _full_spec((1, H)), _full_spec((1, H)), _row_spec(1)],
        out_specs=pl.BlockSpec((G, H), lambda i: (0, 0)),
        out_shape=jax.ShapeDtypeStruct((G, H), jnp.float32),
        scratch_shapes=[pltpu.VMEM((G, H + 1), jnp.float32)],
    )(acc, s, b.reshape(1, H), g.reshape(1, H), bb.reshape(1, H),
      batch.reshape(N, 1).astype(jnp.int32))


E = 800000
_HALF = N // 2
_STRIP = 1568
_NP = 16 * _STRIP
_K = 128
_NECHUNK = E // _K
_NSCHUNK = N // _K
_SELF_TAIL = N - _NSCHUNK * _K
_NCHUNK = _NECHUNK + _NSCHUNK
_ROUNDS = _NCHUNK // 16
_EP = E + N + 16
_SSTRIP = _NP // 16


def _splat(v16, e):
    return lax.gather(
        v16, jnp.full((16, 1), e, jnp.int32),
        dimension_numbers=lax.GatherDimensionNumbers(
            offset_dims=(), collapsed_slice_dims=(0,), start_index_map=(0,)),
        slice_sizes=(1,),
        mode=lax.GatherScatterMode.PROMISE_IN_BOUNDS)


def _attn_body(hs_hbm, hd_hbm, src_hbm, dst_hbm,
               p_out, s_out,
               hs_v, hd_v, src_v, dst_v, dloc_v, p_v, s_sh):
    c = lax.axis_index("c")
    t = lax.axis_index("s")
    base = pl.multiple_of(c * _HALF, 8)
    pbase = pl.multiple_of(c * _EP, 8)
    dummy = _HALF + t
    off = pl.multiple_of(t * _SSTRIP, 32)

    z16f = jnp.zeros((16,), jnp.float32)
    for j in range(_K // 16):
        p_v[pl.ds(j * 16, 16)] = z16f
    for k in range(_SSTRIP // _K):
        pltpu.sync_copy(p_v, s_sh.at[pl.ds(off + k * _K, _K)])
    pltpu.sync_copy(p_v.at[pl.ds(0, _SSTRIP % _K)],
                    s_sh.at[pl.ds(off + (_SSTRIP // _K) * _K, _SSTRIP % _K)])

    pltpu.sync_copy(hs_hbm, hs_v)
    pltpu.sync_copy(hd_hbm.at[pl.ds(base, _HALF)], hd_v.at[pl.ds(0, _HALF)])
    plsc.subcore_barrier()

    def group(j, s16, d16):
        sl = pl.ds(j * 16, 16)
        hsg = plsc.load_gather(hs_v, [s16])
        dl16 = jnp.clip(d16 - base, 0, _HALF - 1)
        hdg = plsc.load_gather(hd_v, [dl16])
        l = hsg + hdg
        p16 = jnp.exp(jnp.where(l >= 0, l, 0.2 * l))
        owned = (d16 >= base) & (d16 < base + _HALF)
        dloc_v[sl] = jnp.where(owned, dl16, dummy)
        p_v[sl] = p16

    def edge_chunk(cc):
        eoff = pl.multiple_of(cc * _K, _K)
        pltpu.sync_copy(src_hbm.at[pl.ds(eoff, _K)], src_v)
        pltpu.sync_copy(dst_hbm.at[pl.ds(eoff, _K)], dst_v)

        def grp(j, carry):
            group(j, src_v[pl.ds(j * 16, 16)], dst_v[pl.ds(j * 16, 16)])
            return carry

        lax.fori_loop(0, _K // 16, grp, 0)
        pltpu.sync_copy(p_v, p_out.at[pl.ds(pbase + eoff, _K)])
        pltpu.sync_copy(p_v, s_sh.at[dloc_v], add=True)

    def self_chunk(cs, sz):
        noff = pl.multiple_of(cs * _K, _K)

        def grp(j, carry):
            n16 = noff + j * 16 + lax.iota(jnp.int32, 16)
            group(j, n16, n16)
            return carry

        lax.fori_loop(0, sz // 16, grp, 0)
        if sz < _K:
            dummy16 = jnp.full((16,), dummy, jnp.int32)
            for j in range(sz // 16, _K // 16):
                dloc_v[pl.ds(j * 16, 16)] = dummy16
        pltpu.sync_copy(p_v.at[pl.ds(0, sz)],
                        p_out.at[pl.ds(pbase + E + noff, sz)])
        pltpu.sync_copy(p_v, s_sh.at[dloc_v], add=True)

    def round_(k, carry):
        cc = k * 16 + t
        lax.cond(cc < _NECHUNK,
                 lambda: edge_chunk(cc),
                 lambda: self_chunk(cc - _NECHUNK, _K))
        return carry

    lax.fori_loop(0, _ROUNDS, round_, 0)

    @pl.when(t == 0)
    def _():
        self_chunk(_NSCHUNK, _SELF_TAIL)

    plsc.subcore_barrier()
    for k in range(_SSTRIP // _K):
        pltpu.sync_copy(s_sh.at[pl.ds(off + k * _K, _K)], p_v)
        pltpu.sync_copy(p_v, s_out.at[pl.ds(c * _NP + off + k * _K, _K)])
    pltpu.sync_copy(s_sh.at[pl.ds(off + (_SSTRIP // _K) * _K, _SSTRIP % _K)],
                    p_v.at[pl.ds(0, _SSTRIP % _K)])
    pltpu.sync_copy(p_v.at[pl.ds(0, _SSTRIP % _K)],
                    s_out.at[pl.ds(c * _NP + off + (_SSTRIP // _K) * _K,
                                   _SSTRIP % _K)])


_attn_sc = functools.partial(
    pl.kernel,
    out_type=[jax.ShapeDtypeStruct((2 * _EP,), jnp.float32),
              jax.ShapeDtypeStruct((2 * _NP,), jnp.float32)],
    mesh=plsc.VectorSubcoreMesh(core_axis_name="c", subcore_axis_name="s"),
    compiler_params=pltpu.CompilerParams(needs_layout_passes=False,
                                         use_tc_tiling_on_sc=False),
    scratch_types=[
        pltpu.VMEM((N,), jnp.float32),
        pltpu.VMEM((_HALF + 24,), jnp.float32),
        pltpu.VMEM((_K,), jnp.int32),
        pltpu.VMEM((_K,), jnp.int32),
        pltpu.VMEM((_K,), jnp.int32),
        pltpu.VMEM((_K,), jnp.float32),
        pltpu.VMEM_SHARED((_NP,), jnp.float32),
    ],
)(_attn_body)


def _rows_body(h_hbm, src_hbm, dst_hbm, p_hbm,
               acc_out,
               src_v, dst_v, dloc_v, p_v, rows_v, sem, acc_sh):
    c = lax.axis_index("c")
    t = lax.axis_index("s")
    base = pl.multiple_of(c * _HALF, 8)
    pbase = pl.multiple_of(c * _EP, 8)
    dummy = _HALF + t
    off = pl.multiple_of(t * _STRIP, 32)

    z16f = jnp.zeros((16,), jnp.float32)

    def zrow(r, carry):
        for q in range(H // 16):
            rows_v[r, pl.ds(q * 16, 16)] = z16f
        return carry

    lax.fori_loop(0, _K, zrow, 0)
    for k in range(12):
        pltpu.sync_copy(rows_v, acc_sh.at[pl.ds(off + k * _K, _K)])
    pltpu.sync_copy(rows_v.at[pl.ds(0, 32)],
                    acc_sh.at[pl.ds(off + 12 * _K, 32)])
    plsc.subcore_barrier()

    def scale(j):
        sl = pl.ds(j * 16, 16)
        d16 = dst_v[sl]
        owned = (d16 >= base) & (d16 < base + _HALF)
        dloc_v[sl] = jnp.where(owned, d16 - base, dummy)
        p16 = p_v[sl]
        for e in range(16):
            pe = _splat(p16, e)
            row = j * 16 + e
            for q in range(H // 16):
                cs = pl.ds(q * 16, 16)
                rows_v[row, cs] = rows_v[row, cs] * pe

    def edge_chunk(cc):
        eoff = pl.multiple_of(cc * _K, _K)
        pltpu.sync_copy(src_hbm.at[pl.ds(eoff, _K)], src_v)
        pltpu.sync_copy(dst_hbm.at[pl.ds(eoff, _K)], dst_v)
        pltpu.sync_copy(p_hbm.at[pl.ds(pbase + eoff, _K)], p_v)
        pltpu.async_copy(h_hbm.at[src_v], rows_v, sem).wait()

        def grp(j, carry):
            scale(j)
            return carry

        lax.fori_loop(0, _K // 16, grp, 0)
        pltpu.sync_copy(rows_v, acc_sh.at[dloc_v], add=True)

    def self_chunk(cs, sz):
        noff = pl.multiple_of(cs * _K, _K)
        pltpu.sync_copy(h_hbm.at[pl.ds(noff, sz)], rows_v.at[pl.ds(0, sz)])
        pltpu.sync_copy(p_hbm.at[pl.ds(pbase + E + noff, sz)],
                        p_v.at[pl.ds(0, sz)])

        def grp(j, carry):
            sl = pl.ds(j * 16, 16)
            dst_v[sl] = noff + j * 16 + lax.iota(jnp.int32, 16)
            scale(j)
            return carry

        lax.fori_loop(0, sz // 16, grp, 0)
        if sz < _K:
            dummy16 = jnp.full((16,), dummy, jnp.int32)
            for j in range(sz // 16, _K // 16):
                dloc_v[pl.ds(j * 16, 16)] = dummy16
        pltpu.sync_copy(rows_v, acc_sh.at[dloc_v], add=True)

    def round_(k, carry):
        cc = k * 16 + t
        lax.cond(cc < _NECHUNK,
                 lambda: edge_chunk(cc),
                 lambda: self_chunk(cc - _NECHUNK, _K))
        return carry

    lax.fori_loop(0, _ROUNDS, round_, 0)

    @pl.when(t == 0)
    def _():
        self_chunk(_NSCHUNK, _SELF_TAIL)

    plsc.subcore_barrier()
    for k in range(12):
        pltpu.sync_copy(acc_sh.at[pl.ds(off + k * _K, _K)], rows_v)
        pltpu.sync_copy(rows_v, acc_out.at[c, pl.ds(off + k * _K, _K)])
    pltpu.sync_copy(acc_sh.at[pl.ds(off + 12 * _K, 32)],
                    rows_v.at[pl.ds(0, 32)])
    pltpu.sync_copy(rows_v.at[pl.ds(0, 32)],
                    acc_out.at[c, pl.ds(off + 12 * _K, 32)])


_rows_sc = functools.partial(
    pl.kernel,
    out_type=[jax.ShapeDtypeStruct((2, _NP, H), jnp.float32)],
    mesh=plsc.VectorSubcoreMesh(core_axis_name="c", subcore_axis_name="s"),
    compiler_params=pltpu.CompilerParams(needs_layout_passes=False,
                                         use_tc_tiling_on_sc=False),
    scratch_types=[
        pltpu.VMEM((_K,), jnp.int32),
        pltpu.VMEM((_K,), jnp.int32),
        pltpu.VMEM((_K,), jnp.int32),
        pltpu.VMEM((_K,), jnp.float32),
        pltpu.VMEM((_K, H), jnp.float32),
        pltpu.SemaphoreType.DMA,
        pltpu.VMEM_SHARED((_NP, H), jnp.float32),
    ],
)(_rows_body)


def _edge_pass(h, hs, hd, src, dst):
    p_all, s_p = _attn_sc(hs.reshape(N), hd.reshape(N), src, dst)
    acc_p, = _rows_sc(h, src, dst, p_all)
    acc = jnp.concatenate([acc_p[0, :_HALF], acc_p[1, :_HALF]])
    s = jnp.concatenate([s_p[:_HALF], s_p[_NP:_NP + _HALF]])
    return acc, s.reshape(N, 1)


def kernel(x, edge_index, edge_attr, global_features, batch,
           W1, as1, ad1, b1, g1, bb1,
           W2, as2, ad2, b2, g2, bb2,
           W3, as3, ad3, b3, g3, bb3):
    src = edge_index[0]
    dst = edge_index[1]

    h, hs, hd = _mm1(x, W1, as1, ad1)
    acc, s = _edge_pass(h, hs, hd, src, dst)

    h, hs, hd = _lnmm(acc, s, b1, g1, bb1, W2, as2, ad2)
    acc, s = _edge_pass(h, hs, hd, src, dst)

    h, hs, hd = _lnmm(acc, s, b2, g2, bb2, W3, as3, ad3)
    acc, s = _edge_pass(h, hs, hd, src, dst)

    return _lnpool(acc, s, b3, g3, bb3, batch)


# speedup vs baseline: 68.0735x; 1.3705x over previous
import functools

import jax
import jax.numpy as jnp
from jax import lax
from jax.experimental import pallas as pl
from jax.experimental.pallas import tpu as pltpu
from jax.experimental.pallas import tpu_sc as plsc

N = 50000
D = 128
H = 64
G = 32
_BLK = 2000


def _stats(h, a_s, a_d):
    hs = jnp.dot(h, a_s, preferred_element_type=jnp.float32)
    hd = jnp.dot(h, a_d, preferred_element_type=jnp.float32)
    return hs, hd


def _mm1_body(x_ref, w_ref, as_ref, ad_ref, h_ref, hs_ref, hd_ref):
    h = jnp.dot(x_ref[...], w_ref[...], preferred_element_type=jnp.float32)
    hs, hd = _stats(h, as_ref[...], ad_ref[...])
    h_ref[...] = h
    hs_ref[...] = hs
    hd_ref[...] = hd


def _ln(y0, g, bb):
    mu = jnp.mean(y0, axis=-1, keepdims=True)
    v = jnp.mean((y0 - mu) ** 2, axis=-1, keepdims=True)
    return (y0 - mu) * jax.lax.rsqrt(v + 1e-5) * g + bb


def _lnmm_body(acc_ref, s_ref, b_ref, g_ref, bb_ref, w_ref, as_ref, ad_ref,
               h_ref, hs_ref, hd_ref):
    y0 = acc_ref[...] / (s_ref[...] + 1e-16) + b_ref[...]
    y = jnp.maximum(_ln(y0, g_ref[...], bb_ref[...]), 0.0)
    h = jnp.dot(y, w_ref[...], preferred_element_type=jnp.float32)
    hs, hd = _stats(h, as_ref[...], ad_ref[...])
    h_ref[...] = h
    hs_ref[...] = hs
    hd_ref[...] = hd


def _lnpool_body(acc_ref, s_ref, b_ref, g_ref, bb_ref, batch_ref,
                 emb_ref, scr_ref):
    i = pl.program_id(0)

    @pl.when(i == 0)
    def _():
        scr_ref[...] = jnp.zeros_like(scr_ref)

    y0 = acc_ref[...] / (s_ref[...] + 1e-16) + b_ref[...]
    y = jnp.maximum(_ln(y0, g_ref[...], bb_ref[...]), 0.0)
    onehot = (batch_ref[...] == jax.lax.broadcasted_iota(jnp.int32, (1, G), 1)
              ).astype(jnp.float32)
    y_aug = jnp.concatenate([y, jnp.ones_like(y[:, :1])], axis=1)
    scr_ref[...] += jax.lax.dot_general(
        onehot, y_aug, (((0,), (0,)), ((), ())),
        preferred_element_type=jnp.float32)

    @pl.when(i == pl.num_programs(0) - 1)
    def _():
        sums = scr_ref[:, :H]
        cnts = jnp.clip(scr_ref[:, H:H + 1], 1.0, None)
        emb_ref[...] = sums / cnts


def _row_spec(width):
    return pl.BlockSpec((_BLK, width), lambda i: (i, 0))


def _full_spec(shape):
    return pl.BlockSpec(shape, lambda i: tuple(0 for _ in shape))


def _mm1(x, W, a_s, a_d):
    grid = (N // _BLK,)
    outs = (
        jax.ShapeDtypeStruct((N, H), jnp.float32),
        jax.ShapeDtypeStruct((N, 1), jnp.float32),
        jax.ShapeDtypeStruct((N, 1), jnp.float32),
    )
    return pl.pallas_call(
        _mm1_body,
        grid=grid,
        in_specs=[_row_spec(D), _full_spec((D, H)), _full_spec((H, 1)),
                  _full_spec((H, 1))],
        out_specs=[_row_spec(H), _row_spec(1), _row_spec(1)],
        out_shape=outs,
    )(x, W, a_s.reshape(H, 1), a_d.reshape(H, 1))


def _lnmm(acc, s, b, g, bb, W, a_s, a_d):
    grid = (N // _BLK,)
    outs = (
        jax.ShapeDtypeStruct((N, H), jnp.float32),
        jax.ShapeDtypeStruct((N, 1), jnp.float32),
        jax.ShapeDtypeStruct((N, 1), jnp.float32),
    )
    return pl.pallas_call(
        _lnmm_body,
        grid=grid,
        in_specs=[_row_spec(H), _row_spec(1), _full_spec((1, H)),
                  _full_spec((1, H)), _full_spec((1, H)), _full_spec((H, H)),
                  _full_spec((H, 1)), _full_spec((H, 1))],
        out_specs=[_row_spec(H), _row_spec(1), _row_spec(1)],
        out_shape=outs,
    )(acc, s, b.reshape(1, H), g.reshape(1, H), bb.reshape(1, H), W,
      a_s.reshape(H, 1), a_d.reshape(H, 1))


def _lnpool(acc, s, b, g, bb, batch):
    grid = (N // _BLK,)
    return pl.pallas_call(
        _lnpool_body,
        grid=grid,
        in_specs=[_row_spec(H), _row_spec(1), _full_spec((1, H)),
                  _full_spec((1, H)), _full_spec((1, H)), _row_spec(1)],
        out_specs=pl.BlockSpec((G, H), lambda i: (0, 0)),
        out_shape=jax.ShapeDtypeStruct((G, H), jnp.float32),
        scratch_shapes=[pltpu.VMEM((G, H + 1), jnp.float32)],
    )(acc, s, b.reshape(1, H), g.reshape(1, H), bb.reshape(1, H),
      batch.reshape(N, 1).astype(jnp.int32))


E = 800000
_HALF = N // 2
_STRIP = 1568
_NP = 16 * _STRIP
_K = 128
_NECHUNK = E // _K
_NSCHUNK = N // _K
_SELF_TAIL = N - _NSCHUNK * _K
_NCHUNK = _NECHUNK + _NSCHUNK
_ROUNDS = _NCHUNK // 16
_EP = E + N + 16
_SSTRIP = _NP // 16
_SK = 4
_NESUP = _NECHUNK // _SK
_NSSUP = _NSCHUNK // _SK


def _splat(v16, e):
    return lax.gather(
        v16, jnp.full((16, 1), e, jnp.int32),
        dimension_numbers=lax.GatherDimensionNumbers(
            offset_dims=(), collapsed_slice_dims=(0,), start_index_map=(0,)),
        slice_sizes=(1,),
        mode=lax.GatherScatterMode.PROMISE_IN_BOUNDS)


def _attn_body(hs_hbm, hd_hbm, src_hbm, dst_hbm,
               p_out, s_out,
               hs_v, hd_v, src_v, dst_v, dloc_v, p_v, s_sh):
    c = lax.axis_index("c")
    t = lax.axis_index("s")
    base = pl.multiple_of(c * _HALF, 8)
    pbase = pl.multiple_of(c * _EP, 8)
    dummy = _HALF + t
    off = pl.multiple_of(t * _SSTRIP, 32)

    z16f = jnp.zeros((16,), jnp.float32)
    for j in range(_K // 16):
        p_v[pl.ds(j * 16, 16)] = z16f
    for k in range(_SSTRIP // _K):
        pltpu.sync_copy(p_v, s_sh.at[pl.ds(off + k * _K, _K)])
    pltpu.sync_copy(p_v.at[pl.ds(0, _SSTRIP % _K)],
                    s_sh.at[pl.ds(off + (_SSTRIP // _K) * _K, _SSTRIP % _K)])

    pltpu.sync_copy(hs_hbm, hs_v)
    pltpu.sync_copy(hd_hbm.at[pl.ds(base, _HALF)], hd_v.at[pl.ds(0, _HALF)])
    plsc.subcore_barrier()

    def group(j, s16, d16):
        sl = pl.ds(j * 16, 16)
        hsg = plsc.load_gather(hs_v, [s16])
        dl16 = jnp.clip(d16 - base, 0, _HALF - 1)
        hdg = plsc.load_gather(hd_v, [dl16])
        l = hsg + hdg
        p16 = jnp.exp(jnp.where(l >= 0, l, 0.2 * l))
        owned = (d16 >= base) & (d16 < base + _HALF)
        dloc_v[sl] = jnp.where(owned, dl16, dummy)
        p_v[sl] = p16

    def edge_chunk(cc):
        eoff = pl.multiple_of(cc * _K, _K)
        pltpu.sync_copy(src_hbm.at[pl.ds(eoff, _K)], src_v)
        pltpu.sync_copy(dst_hbm.at[pl.ds(eoff, _K)], dst_v)

        def grp(j, carry):
            group(j, src_v[pl.ds(j * 16, 16)], dst_v[pl.ds(j * 16, 16)])
            return carry

        lax.fori_loop(0, _K // 16, grp, 0)
        pltpu.sync_copy(p_v, p_out.at[pl.ds(pbase + eoff, _K)])
        pltpu.sync_copy(p_v, s_sh.at[dloc_v], add=True)

    def self_chunk(cs, sz):
        noff = pl.multiple_of(cs * _K, _K)

        def grp(j, carry):
            n16 = noff + j * 16 + lax.iota(jnp.int32, 16)
            group(j, n16, n16)
            return carry

        lax.fori_loop(0, sz // 16, grp, 0)
        if sz < _K:
            dummy16 = jnp.full((16,), dummy, jnp.int32)
            for j in range(sz // 16, _K // 16):
                dloc_v[pl.ds(j * 16, 16)] = dummy16
        pltpu.sync_copy(p_v.at[pl.ds(0, sz)],
                        p_out.at[pl.ds(pbase + E + noff, sz)])
        pltpu.sync_copy(p_v, s_sh.at[dloc_v], add=True)

    def round_(k, carry):
        cc = k * 16 + t
        lax.cond(cc < _NECHUNK,
                 lambda: edge_chunk(cc),
                 lambda: self_chunk(cc - _NECHUNK, _K))
        return carry

    lax.fori_loop(0, _ROUNDS, round_, 0)

    @pl.when(t == 0)
    def _():
        self_chunk(_NSCHUNK, _SELF_TAIL)

    plsc.subcore_barrier()
    for k in range(_SSTRIP // _K):
        pltpu.sync_copy(s_sh.at[pl.ds(off + k * _K, _K)], p_v)
        pltpu.sync_copy(p_v, s_out.at[pl.ds(c * _NP + off + k * _K, _K)])
    pltpu.sync_copy(s_sh.at[pl.ds(off + (_SSTRIP // _K) * _K, _SSTRIP % _K)],
                    p_v.at[pl.ds(0, _SSTRIP % _K)])
    pltpu.sync_copy(p_v.at[pl.ds(0, _SSTRIP % _K)],
                    s_out.at[pl.ds(c * _NP + off + (_SSTRIP // _K) * _K,
                                   _SSTRIP % _K)])


_attn_sc = functools.partial(
    pl.kernel,
    out_type=[jax.ShapeDtypeStruct((2 * _EP,), jnp.float32),
              jax.ShapeDtypeStruct((2 * _NP,), jnp.float32)],
    mesh=plsc.VectorSubcoreMesh(core_axis_name="c", subcore_axis_name="s"),
    compiler_params=pltpu.CompilerParams(needs_layout_passes=False,
                                         use_tc_tiling_on_sc=False),
    scratch_types=[
        pltpu.VMEM((N,), jnp.float32),
        pltpu.VMEM((_HALF + 24,), jnp.float32),
        pltpu.VMEM((_K,), jnp.int32),
        pltpu.VMEM((_K,), jnp.int32),
        pltpu.VMEM((_K,), jnp.int32),
        pltpu.VMEM((_K,), jnp.float32),
        pltpu.VMEM_SHARED((_NP,), jnp.float32),
    ],
)(_attn_body)


def _rows_body(h_hbm, src_hbm, dst_hbm, p_hbm,
               acc_out,
               src4_v, dst4_v, p4_v, dloc4_v, rows0_v, rows1_v,
               gsem0, gsem1, ssem0, ssem1, acc_sh):
    c = lax.axis_index("c")
    t = lax.axis_index("s")
    base = pl.multiple_of(c * _HALF, 8)
    pbase = pl.multiple_of(c * _EP, 8)
    dummy = _HALF + t
    off = pl.multiple_of(t * _STRIP, 32)

    z16f = jnp.zeros((16,), jnp.float32)

    def zrow(r, carry):
        for q in range(H // 16):
            rows0_v[r, pl.ds(q * 16, 16)] = z16f
        return carry

    lax.fori_loop(0, _K, zrow, 0)
    for k in range(12):
        pltpu.sync_copy(rows0_v, acc_sh.at[pl.ds(off + k * _K, _K)])
    pltpu.sync_copy(rows0_v.at[pl.ds(0, 32)],
                    acc_sh.at[pl.ds(off + 12 * _K, 32)])
    plsc.subcore_barrier()

    bufs = (rows0_v, rows1_v)
    gsems = (gsem0, gsem1)
    ssems = (ssem0, ssem1)

    def scale_sub(sub, buf, noff_or_none):
        def grp(j, carry):
            sl = pl.ds(j * 16, 16)
            if noff_or_none is None:
                d16 = dst4_v[pl.ds(sub * _K + j * 16, 16)]
            else:
                d16 = noff_or_none + sub * _K + j * 16 + lax.iota(jnp.int32, 16)
            owned = (d16 >= base) & (d16 < base + _HALF)
            dloc4_v[sub, sl] = jnp.where(owned, d16 - base, dummy)
            p16 = p4_v[pl.ds(sub * _K + j * 16, 16)]
            for e in range(16):
                pe = _splat(p16, e)
                row = j * 16 + e
                for q in range(H // 16):
                    cs = pl.ds(q * 16, 16)
                    buf[row, cs] = buf[row, cs] * pe
            return carry

        lax.fori_loop(0, _K // 16, grp, 0)

    def sup(eoff, nsub, is_self, tail_sz=_K):
        szb = (nsub - 1) * _K + tail_sz
        if is_self:
            pltpu.sync_copy(p_hbm.at[pl.ds(pbase + E + eoff, szb)],
                            p4_v.at[pl.ds(0, szb)])
        else:
            pltpu.sync_copy(src_hbm.at[pl.ds(eoff, szb)],
                            src4_v.at[pl.ds(0, szb)])
            pltpu.sync_copy(dst_hbm.at[pl.ds(eoff, szb)],
                            dst4_v.at[pl.ds(0, szb)])
            pltpu.sync_copy(p_hbm.at[pl.ds(pbase + eoff, szb)],
                            p4_v.at[pl.ds(0, szb)])

        def start_gather(sub):
            b = bufs[sub % 2]
            g = gsems[sub % 2]
            sz = tail_sz if sub == nsub - 1 else _K
            if is_self:
                return pltpu.async_copy(
                    h_hbm.at[pl.ds(eoff + sub * _K, sz)],
                    b.at[pl.ds(0, sz)], g)
            return pltpu.async_copy(
                h_hbm.at[src4_v.at[pl.ds(sub * _K, _K)]], b, gsems[sub % 2])

        pend_g = start_gather(0)
        pend_s = [None, None]
        for sub in range(nsub):
            buf = bufs[sub % 2]
            pend_g.wait()
            if sub + 1 < nsub:
                if pend_s[(sub + 1) % 2] is not None:
                    pend_s[(sub + 1) % 2].wait()
                    pend_s[(sub + 1) % 2] = None
                pend_g = start_gather(sub + 1)
            if is_self and sub == nsub - 1 and tail_sz < _K:
                dummy16 = jnp.full((16,), dummy, jnp.int32)
                for j in range(tail_sz // 16, _K // 16):
                    dloc4_v[sub, pl.ds(j * 16, 16)] = dummy16
                ngrp = tail_sz // 16
                def tgrp(j, carry):
                    sl = pl.ds(j * 16, 16)
                    d16 = eoff + sub * _K + j * 16 + lax.iota(jnp.int32, 16)
                    owned = (d16 >= base) & (d16 < base + _HALF)
                    dloc4_v[sub, sl] = jnp.where(owned, d16 - base, dummy)
                    p16 = p4_v[pl.ds(sub * _K + j * 16, 16)]
                    for e in range(16):
                        pe = _splat(p16, e)
                        row = j * 16 + e
                        for q in range(H // 16):
                            cs = pl.ds(q * 16, 16)
                            buf[row, cs] = buf[row, cs] * pe
                    return carry
                lax.fori_loop(0, ngrp, tgrp, 0)
            else:
                scale_sub(sub, buf, eoff if is_self else None)
            if pend_s[sub % 2] is not None:
                pend_s[sub % 2].wait()
            pend_s[sub % 2] = pltpu.async_copy(
                buf, acc_sh.at[dloc4_v.at[sub]], ssems[sub % 2], add=True)
        for d in pend_s:
            if d is not None:
                d.wait()

    _SKB = _SK * _K

    def eround(k, carry):
        sup((k * 16 + t) * _SKB, _SK, False)
        return carry

    lax.fori_loop(0, _NESUP // 16, eround, 0)

    def sround(k, carry):
        sup((k * 16 + t) * _SKB, _SK, True)
        return carry

    lax.fori_loop(0, _NSSUP // 16, sround, 0)

    @pl.when(t < _NESUP % 16)
    def _():
        sup(((_NESUP // 16) * 16 + t) * _SKB, _SK, False)

    @pl.when(t == 10)
    def _():
        sup((_NSSUP - 1) * _SKB, _SK, True)

    @pl.when(t == 11)
    def _():
        sup(_NESUP * _SKB, 1, False)

    @pl.when(t == 12)
    def _():
        sup(_NESUP * _SKB + _K, 1, False)

    @pl.when(t == 13)
    def _():
        sup(_NSSUP * _SKB, 1, True)

    @pl.when(t == 14)
    def _():
        sup(_NSSUP * _SKB + _K, 1, True)

    @pl.when(t == 15)
    def _():
        sup(_NSSUP * _SKB + 2 * _K, 1, True, tail_sz=_SELF_TAIL)

    plsc.subcore_barrier()
    for k in range(12):
        pltpu.sync_copy(acc_sh.at[pl.ds(off + k * _K, _K)], rows0_v)
        pltpu.sync_copy(rows0_v, acc_out.at[c, pl.ds(off + k * _K, _K)])
    pltpu.sync_copy(acc_sh.at[pl.ds(off + 12 * _K, 32)],
                    rows0_v.at[pl.ds(0, 32)])
    pltpu.sync_copy(rows0_v.at[pl.ds(0, 32)],
                    acc_out.at[c, pl.ds(off + 12 * _K, 32)])


_rows_sc = functools.partial(
    pl.kernel,
    out_type=[jax.ShapeDtypeStruct((2, _NP, H), jnp.float32)],
    mesh=plsc.VectorSubcoreMesh(core_axis_name="c", subcore_axis_name="s"),
    compiler_params=pltpu.CompilerParams(needs_layout_passes=False,
                                         use_tc_tiling_on_sc=False),
    scratch_types=[
        pltpu.VMEM((_SK * _K,), jnp.int32),
        pltpu.VMEM((_SK * _K,), jnp.int32),
        pltpu.VMEM((_SK * _K,), jnp.float32),
        pltpu.VMEM((_SK, _K), jnp.int32),
        pltpu.VMEM((_K, H), jnp.float32),
        pltpu.VMEM((_K, H), jnp.float32),
        pltpu.SemaphoreType.DMA,
        pltpu.SemaphoreType.DMA,
        pltpu.SemaphoreType.DMA,
        pltpu.SemaphoreType.DMA,
        pltpu.VMEM_SHARED((_NP, H), jnp.float32),
    ],
)(_rows_body)


def _edge_pass(h, hs, hd, src, dst):
    p_all, s_p = _attn_sc(hs.reshape(N), hd.reshape(N), src, dst)
    acc_p, = _rows_sc(h, src, dst, p_all)
    acc = jnp.concatenate([acc_p[0, :_HALF], acc_p[1, :_HALF]])
    s = jnp.concatenate([s_p[:_HALF], s_p[_NP:_NP + _HALF]])
    return acc, s.reshape(N, 1)


def kernel(x, edge_index, edge_attr, global_features, batch,
           W1, as1, ad1, b1, g1, bb1,
           W2, as2, ad2, b2, g2, bb2,
           W3, as3, ad3, b3, g3, bb3):
    src = edge_index[0]
    dst = edge_index[1]

    h, hs, hd = _mm1(x, W1, as1, ad1)
    acc, s = _edge_pass(h, hs, hd, src, dst)

    h, hs, hd = _lnmm(acc, s, b1, g1, bb1, W2, as2, ad2)
    acc, s = _edge_pass(h, hs, hd, src, dst)

    h, hs, hd = _lnmm(acc, s, b2, g2, bb2, W3, as3, ad3)
    acc, s = _edge_pass(h, hs, hd, src, dst)

    return _lnpool(acc, s, b3, g3, bb3, batch)


# speedup vs baseline: 80.9419x; 1.1890x over previous
import functools

import jax
import jax.numpy as jnp
from jax import lax
from jax.experimental import pallas as pl
from jax.experimental.pallas import tpu as pltpu
from jax.experimental.pallas import tpu_sc as plsc

N = 50000
D = 128
H = 64
G = 32
_BLK = 2000


def _stats(h, a_s, a_d):
    hs = jnp.dot(h, a_s, preferred_element_type=jnp.float32)
    hd = jnp.dot(h, a_d, preferred_element_type=jnp.float32)
    return hs, hd


def _mm1_body(x_ref, w_ref, as_ref, ad_ref, h_ref, hs_ref, hd_ref):
    h = jnp.dot(x_ref[...], w_ref[...], preferred_element_type=jnp.float32)
    hs, hd = _stats(h, as_ref[...], ad_ref[...])
    h_ref[...] = h
    hs_ref[...] = hs
    hd_ref[...] = hd


def _ln(y0, g, bb):
    mu = jnp.mean(y0, axis=-1, keepdims=True)
    v = jnp.mean((y0 - mu) ** 2, axis=-1, keepdims=True)
    return (y0 - mu) * jax.lax.rsqrt(v + 1e-5) * g + bb


def _lnmm_body(acc_ref, s_ref, b_ref, g_ref, bb_ref, w_ref, as_ref, ad_ref,
               h_ref, hs_ref, hd_ref):
    y0 = acc_ref[...] / (s_ref[...] + 1e-16) + b_ref[...]
    y = jnp.maximum(_ln(y0, g_ref[...], bb_ref[...]), 0.0)
    h = jnp.dot(y, w_ref[...], preferred_element_type=jnp.float32)
    hs, hd = _stats(h, as_ref[...], ad_ref[...])
    h_ref[...] = h
    hs_ref[...] = hs
    hd_ref[...] = hd


def _lnpool_body(acc_ref, s_ref, b_ref, g_ref, bb_ref, batch_ref,
                 emb_ref, scr_ref):
    i = pl.program_id(0)

    @pl.when(i == 0)
    def _():
        scr_ref[...] = jnp.zeros_like(scr_ref)

    y0 = acc_ref[...] / (s_ref[...] + 1e-16) + b_ref[...]
    y = jnp.maximum(_ln(y0, g_ref[...], bb_ref[...]), 0.0)
    onehot = (batch_ref[...] == jax.lax.broadcasted_iota(jnp.int32, (1, G), 1)
              ).astype(jnp.float32)
    y_aug = jnp.concatenate([y, jnp.ones_like(y[:, :1])], axis=1)
    scr_ref[...] += jax.lax.dot_general(
        onehot, y_aug, (((0,), (0,)), ((), ())),
        preferred_element_type=jnp.float32)

    @pl.when(i == pl.num_programs(0) - 1)
    def _():
        sums = scr_ref[:, :H]
        cnts = jnp.clip(scr_ref[:, H:H + 1], 1.0, None)
        emb_ref[...] = sums / cnts


def _row_spec(width):
    return pl.BlockSpec((_BLK, width), lambda i: (i, 0))


def _full_spec(shape):
    return pl.BlockSpec(shape, lambda i: tuple(0 for _ in shape))


def _mm1(x, W, a_s, a_d):
    grid = (N // _BLK,)
    outs = (
        jax.ShapeDtypeStruct((N, H), jnp.float32),
        jax.ShapeDtypeStruct((N, 1), jnp.float32),
        jax.ShapeDtypeStruct((N, 1), jnp.float32),
    )
    return pl.pallas_call(
        _mm1_body,
        grid=grid,
        in_specs=[_row_spec(D), _full_spec((D, H)), _full_spec((H, 1)),
                  _full_spec((H, 1))],
        out_specs=[_row_spec(H), _row_spec(1), _row_spec(1)],
        out_shape=outs,
    )(x, W, a_s.reshape(H, 1), a_d.reshape(H, 1))


def _lnmm(acc, s, b, g, bb, W, a_s, a_d):
    grid = (N // _BLK,)
    outs = (
        jax.ShapeDtypeStruct((N, H), jnp.float32),
        jax.ShapeDtypeStruct((N, 1), jnp.float32),
        jax.ShapeDtypeStruct((N, 1), jnp.float32),
    )
    return pl.pallas_call(
        _lnmm_body,
        grid=grid,
        in_specs=[_row_spec(H), _row_spec(1), _full_spec((1, H)),
                  _full_spec((1, H)), _full_spec((1, H)), _full_spec((H, H)),
                  _full_spec((H, 1)), _full_spec((H, 1))],
        out_specs=[_row_spec(H), _row_spec(1), _row_spec(1)],
        out_shape=outs,
    )(acc, s, b.reshape(1, H), g.reshape(1, H), bb.reshape(1, H), W,
      a_s.reshape(H, 1), a_d.reshape(H, 1))


def _lnpool(acc, s, b, g, bb, batch):
    grid = (N // _BLK,)
    return pl.pallas_call(
        _lnpool_body,
        grid=grid,
        in_specs=[_row_spec(H), _row_spec(1), _full_spec((1, H)),
                  _full_spec((1, H)), _full_spec((1, H)), _row_spec(1)],
        out_specs=pl.BlockSpec((G, H), lambda i: (0, 0)),
        out_shape=jax.ShapeDtypeStruct((G, H), jnp.float32),
        scratch_shapes=[pltpu.VMEM((G, H + 1), jnp.float32)],
    )(acc, s, b.reshape(1, H), g.reshape(1, H), bb.reshape(1, H),
      batch.reshape(N, 1).astype(jnp.int32))


E = 800000
_HALF = N // 2
_STRIP = 1568
_NP = 16 * _STRIP
_K = 128
_NECHUNK = E // _K
_NSCHUNK = N // _K
_SELF_TAIL = N - _NSCHUNK * _K
_NCHUNK = _NECHUNK + _NSCHUNK
_ROUNDS = _NCHUNK // 16
_EP = E + N + 16
_SSTRIP = _NP // 16
_SK = 4
_NESUP = _NECHUNK // _SK
_NSSUP = _NSCHUNK // _SK


def _splat(v16, e):
    return lax.gather(
        v16, jnp.full((16, 1), e, jnp.int32),
        dimension_numbers=lax.GatherDimensionNumbers(
            offset_dims=(), collapsed_slice_dims=(0,), start_index_map=(0,)),
        slice_sizes=(1,),
        mode=lax.GatherScatterMode.PROMISE_IN_BOUNDS)


def _attn_body(hs_hbm, hd_hbm, src_hbm, dst_hbm,
               p_out, s_out,
               hs_v, hd_v, src4_v, dst4_v, dloc4_v, p4_v,
               ssemA, ssemB, acc_unused, s_sh):
    c = lax.axis_index("c")
    t = lax.axis_index("s")
    base = pl.multiple_of(c * _HALF, 8)
    pbase = pl.multiple_of(c * _EP, 8)
    dummy = _HALF + t
    off = pl.multiple_of(t * _SSTRIP, 32)

    z16f = jnp.zeros((16,), jnp.float32)
    for j in range(_K // 16):
        p4_v[pl.ds(j * 16, 16)] = z16f
    for k in range(_SSTRIP // _K):
        pltpu.sync_copy(p4_v.at[pl.ds(0, _K)],
                        s_sh.at[pl.ds(off + k * _K, _K)])
    pltpu.sync_copy(p4_v.at[pl.ds(0, _SSTRIP % _K)],
                    s_sh.at[pl.ds(off + (_SSTRIP // _K) * _K, _SSTRIP % _K)])

    pltpu.sync_copy(hs_hbm, hs_v)
    pltpu.sync_copy(hd_hbm.at[pl.ds(base, _HALF)], hd_v.at[pl.ds(0, _HALF)])
    plsc.subcore_barrier()

    ssems = (ssemA, ssemB)

    def sup(eoff, nsub, is_self, tail_sz=_K):
        szb = (nsub - 1) * _K + tail_sz
        if not is_self:
            pltpu.sync_copy(src_hbm.at[pl.ds(eoff, szb)],
                            src4_v.at[pl.ds(0, szb)])
            pltpu.sync_copy(dst_hbm.at[pl.ds(eoff, szb)],
                            dst4_v.at[pl.ds(0, szb)])
        pend = [None, None]
        for sub in range(nsub):
            sz = tail_sz if sub == nsub - 1 else _K

            def grp(j, carry):
                sl = pl.ds(j * 16, 16)
                if is_self:
                    s16 = eoff + sub * _K + j * 16 + lax.iota(jnp.int32, 16)
                    d16 = s16
                else:
                    s16 = src4_v[pl.ds(sub * _K + j * 16, 16)]
                    d16 = dst4_v[pl.ds(sub * _K + j * 16, 16)]
                hsg = plsc.load_gather(hs_v, [s16])
                dl16 = jnp.clip(d16 - base, 0, _HALF - 1)
                hdg = plsc.load_gather(hd_v, [dl16])
                l = hsg + hdg
                p16 = jnp.exp(jnp.where(l >= 0, l, 0.2 * l))
                owned = (d16 >= base) & (d16 < base + _HALF)
                dloc4_v[sub, sl] = jnp.where(owned, dl16, dummy)
                p4_v[pl.ds(sub * _K + j * 16, 16)] = p16
                return carry

            lax.fori_loop(0, sz // 16, grp, 0)
            if sz < _K:
                dummy16 = jnp.full((16,), dummy, jnp.int32)
                for j in range(sz // 16, _K // 16):
                    dloc4_v[sub, pl.ds(j * 16, 16)] = dummy16
            if pend[sub % 2] is not None:
                pend[sub % 2].wait()
            pend[sub % 2] = pltpu.async_copy(
                p4_v.at[pl.ds(sub * _K, _K)], s_sh.at[dloc4_v.at[sub]],
                ssems[sub % 2], add=True)
        for d in pend:
            if d is not None:
                d.wait()
        poff = pbase + (E + eoff if is_self else eoff)
        pltpu.sync_copy(p4_v.at[pl.ds(0, szb)], p_out.at[pl.ds(poff, szb)])

    _SKB2 = _SK * _K

    def eround(k, carry):
        sup((k * 16 + t) * _SKB2, _SK, False)
        return carry

    lax.fori_loop(0, _NESUP // 16, eround, 0)

    def sround(k, carry):
        sup((k * 16 + t) * _SKB2, _SK, True)
        return carry

    lax.fori_loop(0, _NSSUP // 16, sround, 0)

    @pl.when(t < _NESUP % 16)
    def _():
        sup(((_NESUP // 16) * 16 + t) * _SKB2, _SK, False)

    @pl.when(t == 10)
    def _():
        sup((_NSSUP - 1) * _SKB2, _SK, True)

    @pl.when(t == 11)
    def _():
        sup(_NESUP * _SKB2, 1, False)

    @pl.when(t == 12)
    def _():
        sup(_NESUP * _SKB2 + _K, 1, False)

    @pl.when(t == 13)
    def _():
        sup(_NSSUP * _SKB2, 1, True)

    @pl.when(t == 14)
    def _():
        sup(_NSSUP * _SKB2 + _K, 1, True)

    @pl.when(t == 15)
    def _():
        sup(_NSSUP * _SKB2 + 2 * _K, 1, True, tail_sz=_SELF_TAIL)

    plsc.subcore_barrier()
    for k in range(_SSTRIP // _K):
        pltpu.sync_copy(s_sh.at[pl.ds(off + k * _K, _K)],
                        p4_v.at[pl.ds(0, _K)])
        pltpu.sync_copy(p4_v.at[pl.ds(0, _K)],
                        s_out.at[pl.ds(c * _NP + off + k * _K, _K)])
    pltpu.sync_copy(s_sh.at[pl.ds(off + (_SSTRIP // _K) * _K, _SSTRIP % _K)],
                    p4_v.at[pl.ds(0, _SSTRIP % _K)])
    pltpu.sync_copy(p4_v.at[pl.ds(0, _SSTRIP % _K)],
                    s_out.at[pl.ds(c * _NP + off + (_SSTRIP // _K) * _K,
                                   _SSTRIP % _K)])


_attn_sc = functools.partial(
    pl.kernel,
    out_type=[jax.ShapeDtypeStruct((2 * _EP,), jnp.float32),
              jax.ShapeDtypeStruct((2 * _NP,), jnp.float32)],
    mesh=plsc.VectorSubcoreMesh(core_axis_name="c", subcore_axis_name="s"),
    compiler_params=pltpu.CompilerParams(needs_layout_passes=False,
                                         use_tc_tiling_on_sc=False),
    scratch_types=[
        pltpu.VMEM((N,), jnp.float32),
        pltpu.VMEM((_HALF + 24,), jnp.float32),
        pltpu.VMEM((_SK * _K,), jnp.int32),
        pltpu.VMEM((_SK * _K,), jnp.int32),
        pltpu.VMEM((_SK, _K), jnp.int32),
        pltpu.VMEM((_SK * _K,), jnp.float32),
        pltpu.SemaphoreType.DMA,
        pltpu.SemaphoreType.DMA,
        pltpu.SemaphoreType.DMA,
        pltpu.VMEM_SHARED((_NP,), jnp.float32),
    ],
)(_attn_body)


def _rows_body(h_hbm, src_hbm, dst_hbm, p_hbm,
               acc_out,
               src4_v, dst4_v, p4_v, dloc4_v, rows0_v, rows1_v,
               gsem0, gsem1, ssem0, ssem1, acc_sh):
    c = lax.axis_index("c")
    t = lax.axis_index("s")
    base = pl.multiple_of(c * _HALF, 8)
    pbase = pl.multiple_of(c * _EP, 8)
    dummy = _HALF + t
    off = pl.multiple_of(t * _STRIP, 32)

    z16f = jnp.zeros((16,), jnp.float32)

    def zrow(r, carry):
        for q in range(H // 16):
            rows0_v[r, pl.ds(q * 16, 16)] = z16f
        return carry

    lax.fori_loop(0, _K, zrow, 0)
    for k in range(12):
        pltpu.sync_copy(rows0_v, acc_sh.at[pl.ds(off + k * _K, _K)])
    pltpu.sync_copy(rows0_v.at[pl.ds(0, 32)],
                    acc_sh.at[pl.ds(off + 12 * _K, 32)])
    plsc.subcore_barrier()

    bufs = (rows0_v, rows1_v)
    gsems = (gsem0, gsem1)
    ssems = (ssem0, ssem1)

    def scale_sub(sub, buf, noff_or_none):
        def grp(j, carry):
            sl = pl.ds(j * 16, 16)
            if noff_or_none is None:
                d16 = dst4_v[pl.ds(sub * _K + j * 16, 16)]
            else:
                d16 = noff_or_none + sub * _K + j * 16 + lax.iota(jnp.int32, 16)
            owned = (d16 >= base) & (d16 < base + _HALF)
            dloc4_v[sub, sl] = jnp.where(owned, d16 - base, dummy)
            p16 = p4_v[pl.ds(sub * _K + j * 16, 16)]
            for e in range(16):
                pe = _splat(p16, e)
                row = j * 16 + e
                for q in range(H // 16):
                    cs = pl.ds(q * 16, 16)
                    buf[row, cs] = buf[row, cs] * pe
            return carry

        lax.fori_loop(0, _K // 16, grp, 0)

    def sup(eoff, nsub, is_self, tail_sz=_K):
        szb = (nsub - 1) * _K + tail_sz
        if is_self:
            pltpu.sync_copy(p_hbm.at[pl.ds(pbase + E + eoff, szb)],
                            p4_v.at[pl.ds(0, szb)])
        else:
            pltpu.sync_copy(src_hbm.at[pl.ds(eoff, szb)],
                            src4_v.at[pl.ds(0, szb)])
            pltpu.sync_copy(dst_hbm.at[pl.ds(eoff, szb)],
                            dst4_v.at[pl.ds(0, szb)])
            pltpu.sync_copy(p_hbm.at[pl.ds(pbase + eoff, szb)],
                            p4_v.at[pl.ds(0, szb)])

        def start_gather(sub):
            b = bufs[sub % 2]
            g = gsems[sub % 2]
            sz = tail_sz if sub == nsub - 1 else _K
            if is_self:
                return pltpu.async_copy(
                    h_hbm.at[pl.ds(eoff + sub * _K, sz)],
                    b.at[pl.ds(0, sz)], g)
            return pltpu.async_copy(
                h_hbm.at[src4_v.at[pl.ds(sub * _K, _K)]], b, gsems[sub % 2])

        pend_g = start_gather(0)
        pend_s = [None, None]
        for sub in range(nsub):
            buf = bufs[sub % 2]
            pend_g.wait()
            if sub + 1 < nsub:
                if pend_s[(sub + 1) % 2] is not None:
                    pend_s[(sub + 1) % 2].wait()
                    pend_s[(sub + 1) % 2] = None
                pend_g = start_gather(sub + 1)
            if is_self and sub == nsub - 1 and tail_sz < _K:
                dummy16 = jnp.full((16,), dummy, jnp.int32)
                for j in range(tail_sz // 16, _K // 16):
                    dloc4_v[sub, pl.ds(j * 16, 16)] = dummy16
                ngrp = tail_sz // 16
                def tgrp(j, carry):
                    sl = pl.ds(j * 16, 16)
                    d16 = eoff + sub * _K + j * 16 + lax.iota(jnp.int32, 16)
                    owned = (d16 >= base) & (d16 < base + _HALF)
                    dloc4_v[sub, sl] = jnp.where(owned, d16 - base, dummy)
                    p16 = p4_v[pl.ds(sub * _K + j * 16, 16)]
                    for e in range(16):
                        pe = _splat(p16, e)
                        row = j * 16 + e
                        for q in range(H // 16):
                            cs = pl.ds(q * 16, 16)
                            buf[row, cs] = buf[row, cs] * pe
                    return carry
                lax.fori_loop(0, ngrp, tgrp, 0)
            else:
                scale_sub(sub, buf, eoff if is_self else None)
            if pend_s[sub % 2] is not None:
                pend_s[sub % 2].wait()
            pend_s[sub % 2] = pltpu.async_copy(
                buf, acc_sh.at[dloc4_v.at[sub]], ssems[sub % 2], add=True)
        for d in pend_s:
            if d is not None:
                d.wait()

    _SKB = _SK * _K

    def eround(k, carry):
        sup((k * 16 + t) * _SKB, _SK, False)
        return carry

    lax.fori_loop(0, _NESUP // 16, eround, 0)

    def sround(k, carry):
        sup((k * 16 + t) * _SKB, _SK, True)
        return carry

    lax.fori_loop(0, _NSSUP // 16, sround, 0)

    @pl.when(t < _NESUP % 16)
    def _():
        sup(((_NESUP // 16) * 16 + t) * _SKB, _SK, False)

    @pl.when(t == 10)
    def _():
        sup((_NSSUP - 1) * _SKB, _SK, True)

    @pl.when(t == 11)
    def _():
        sup(_NESUP * _SKB, 1, False)

    @pl.when(t == 12)
    def _():
        sup(_NESUP * _SKB + _K, 1, False)

    @pl.when(t == 13)
    def _():
        sup(_NSSUP * _SKB, 1, True)

    @pl.when(t == 14)
    def _():
        sup(_NSSUP * _SKB + _K, 1, True)

    @pl.when(t == 15)
    def _():
        sup(_NSSUP * _SKB + 2 * _K, 1, True, tail_sz=_SELF_TAIL)

    plsc.subcore_barrier()
    for k in range(12):
        pltpu.sync_copy(acc_sh.at[pl.ds(off + k * _K, _K)], rows0_v)
        pltpu.sync_copy(rows0_v, acc_out.at[c, pl.ds(off + k * _K, _K)])
    pltpu.sync_copy(acc_sh.at[pl.ds(off + 12 * _K, 32)],
                    rows0_v.at[pl.ds(0, 32)])
    pltpu.sync_copy(rows0_v.at[pl.ds(0, 32)],
                    acc_out.at[c, pl.ds(off + 12 * _K, 32)])


_rows_sc = functools.partial(
    pl.kernel,
    out_type=[jax.ShapeDtypeStruct((2, _NP, H), jnp.float32)],
    mesh=plsc.VectorSubcoreMesh(core_axis_name="c", subcore_axis_name="s"),
    compiler_params=pltpu.CompilerParams(needs_layout_passes=False,
                                         use_tc_tiling_on_sc=False),
    scratch_types=[
        pltpu.VMEM((_SK * _K,), jnp.int32),
        pltpu.VMEM((_SK * _K,), jnp.int32),
        pltpu.VMEM((_SK * _K,), jnp.float32),
        pltpu.VMEM((_SK, _K), jnp.int32),
        pltpu.VMEM((_K, H), jnp.float32),
        pltpu.VMEM((_K, H), jnp.float32),
        pltpu.SemaphoreType.DMA,
        pltpu.SemaphoreType.DMA,
        pltpu.SemaphoreType.DMA,
        pltpu.SemaphoreType.DMA,
        pltpu.VMEM_SHARED((_NP, H), jnp.float32),
    ],
)(_rows_body)


def _edge_pass(h, hs, hd, src, dst):
    p_all, s_p = _attn_sc(hs.reshape(N), hd.reshape(N), src, dst)
    acc_p, = _rows_sc(h, src, dst, p_all)
    acc = jnp.concatenate([acc_p[0, :_HALF], acc_p[1, :_HALF]])
    s = jnp.concatenate([s_p[:_HALF], s_p[_NP:_NP + _HALF]])
    return acc, s.reshape(N, 1)


def kernel(x, edge_index, edge_attr, global_features, batch,
           W1, as1, ad1, b1, g1, bb1,
           W2, as2, ad2, b2, g2, bb2,
           W3, as3, ad3, b3, g3, bb3):
    src = edge_index[0]
    dst = edge_index[1]

    h, hs, hd = _mm1(x, W1, as1, ad1)
    acc, s = _edge_pass(h, hs, hd, src, dst)

    h, hs, hd = _lnmm(acc, s, b1, g1, bb1, W2, as2, ad2)
    acc, s = _edge_pass(h, hs, hd, src, dst)

    h, hs, hd = _lnmm(acc, s, b2, g2, bb2, W3, as3, ad3)
    acc, s = _edge_pass(h, hs, hd, src, dst)

    return _lnpool(acc, s, b3, g3, bb3, batch)


# speedup vs baseline: 89.3114x; 1.1034x over previous
import functools

import jax
import jax.numpy as jnp
from jax import lax
from jax.experimental import pallas as pl
from jax.experimental.pallas import tpu as pltpu
from jax.experimental.pallas import tpu_sc as plsc

N = 50000
D = 128
H = 64
G = 32
_BLK = 2000


def _stats(h, a_s, a_d):
    hs = jnp.dot(h, a_s, preferred_element_type=jnp.float32)
    hd = jnp.dot(h, a_d, preferred_element_type=jnp.float32)
    return hs, hd


def _mm1_body(x_ref, w_ref, as_ref, ad_ref, h_ref, hs_ref, hd_ref):
    h = jnp.dot(x_ref[...], w_ref[...], preferred_element_type=jnp.float32)
    hs, hd = _stats(h, as_ref[...], ad_ref[...])
    h_ref[...] = h
    hs_ref[...] = hs
    hd_ref[...] = hd


def _ln(y0, g, bb):
    mu = jnp.mean(y0, axis=-1, keepdims=True)
    v = jnp.mean((y0 - mu) ** 2, axis=-1, keepdims=True)
    return (y0 - mu) * jax.lax.rsqrt(v + 1e-5) * g + bb


def _lnmm_body(acc_ref, s_ref, b_ref, g_ref, bb_ref, w_ref, as_ref, ad_ref,
               h_ref, hs_ref, hd_ref):
    y0 = acc_ref[...] / (s_ref[...] + 1e-16) + b_ref[...]
    y = jnp.maximum(_ln(y0, g_ref[...], bb_ref[...]), 0.0)
    h = jnp.dot(y, w_ref[...], preferred_element_type=jnp.float32)
    hs, hd = _stats(h, as_ref[...], ad_ref[...])
    h_ref[...] = h
    hs_ref[...] = hs
    hd_ref[...] = hd


def _lnpool_body(acc_ref, s_ref, b_ref, g_ref, bb_ref, batch_ref,
                 emb_ref, scr_ref):
    i = pl.program_id(0)

    @pl.when(i == 0)
    def _():
        scr_ref[...] = jnp.zeros_like(scr_ref)

    y0 = acc_ref[...] / (s_ref[...] + 1e-16) + b_ref[...]
    y = jnp.maximum(_ln(y0, g_ref[...], bb_ref[...]), 0.0)
    onehot = (batch_ref[...] == jax.lax.broadcasted_iota(jnp.int32, (1, G), 1)
              ).astype(jnp.float32)
    y_aug = jnp.concatenate([y, jnp.ones_like(y[:, :1])], axis=1)
    scr_ref[...] += jax.lax.dot_general(
        onehot, y_aug, (((0,), (0,)), ((), ())),
        preferred_element_type=jnp.float32)

    @pl.when(i == pl.num_programs(0) - 1)
    def _():
        sums = scr_ref[:, :H]
        cnts = jnp.clip(scr_ref[:, H:H + 1], 1.0, None)
        emb_ref[...] = sums / cnts


def _row_spec(width):
    return pl.BlockSpec((_BLK, width), lambda i: (i, 0))


def _full_spec(shape):
    return pl.BlockSpec(shape, lambda i: tuple(0 for _ in shape))


def _mm1(x, W, a_s, a_d):
    grid = (N // _BLK,)
    outs = (
        jax.ShapeDtypeStruct((N, H), jnp.float32),
        jax.ShapeDtypeStruct((N, 1), jnp.float32),
        jax.ShapeDtypeStruct((N, 1), jnp.float32),
    )
    return pl.pallas_call(
        _mm1_body,
        grid=grid,
        in_specs=[_row_spec(D), _full_spec((D, H)), _full_spec((H, 1)),
                  _full_spec((H, 1))],
        out_specs=[_row_spec(H), _row_spec(1), _row_spec(1)],
        out_shape=outs,
    )(x, W, a_s.reshape(H, 1), a_d.reshape(H, 1))


def _lnmm(acc, s, b, g, bb, W, a_s, a_d):
    grid = (N // _BLK,)
    outs = (
        jax.ShapeDtypeStruct((N, H), jnp.float32),
        jax.ShapeDtypeStruct((N, 1), jnp.float32),
        jax.ShapeDtypeStruct((N, 1), jnp.float32),
    )
    return pl.pallas_call(
        _lnmm_body,
        grid=grid,
        in_specs=[_row_spec(H), _row_spec(1), _full_spec((1, H)),
                  _full_spec((1, H)), _full_spec((1, H)), _full_spec((H, H)),
                  _full_spec((H, 1)), _full_spec((H, 1))],
        out_specs=[_row_spec(H), _row_spec(1), _row_spec(1)],
        out_shape=outs,
    )(acc, s, b.reshape(1, H), g.reshape(1, H), bb.reshape(1, H), W,
      a_s.reshape(H, 1), a_d.reshape(H, 1))


def _lnpool(acc, s, b, g, bb, batch):
    grid = (N // _BLK,)
    return pl.pallas_call(
        _lnpool_body,
        grid=grid,
        in_specs=[_row_spec(H), _row_spec(1), _full_spec((1, H)),
                  _full_spec((1, H)), _full_spec((1, H)), _row_spec(1)],
        out_specs=pl.BlockSpec((G, H), lambda i: (0, 0)),
        out_shape=jax.ShapeDtypeStruct((G, H), jnp.float32),
        scratch_shapes=[pltpu.VMEM((G, H + 1), jnp.float32)],
    )(acc, s, b.reshape(1, H), g.reshape(1, H), bb.reshape(1, H),
      batch.reshape(N, 1).astype(jnp.int32))


E = 800000
_HALF = N // 2
_STRIP = 1568
_NP = 16 * _STRIP
_K = 128
_NECHUNK = E // _K
_NSCHUNK = N // _K
_SELF_TAIL = N - _NSCHUNK * _K
_NCHUNK = _NECHUNK + _NSCHUNK
_ROUNDS = _NCHUNK // 16
_EP = E + N + 16
_SSTRIP = _NP // 16
_SK = 8
_NESUP = _NECHUNK // _SK
_NSSUP = _NSCHUNK // _SK


def _splat(v16, e):
    return lax.gather(
        v16, jnp.full((16, 1), e, jnp.int32),
        dimension_numbers=lax.GatherDimensionNumbers(
            offset_dims=(), collapsed_slice_dims=(0,), start_index_map=(0,)),
        slice_sizes=(1,),
        mode=lax.GatherScatterMode.PROMISE_IN_BOUNDS)


def _attn_body(hs_hbm, hd_hbm, src_hbm, dst_hbm,
               p_out, s_out,
               hs_v, hd_v, src4_v, dst4_v, dloc4_v, p4_v,
               ssemA, ssemB, acc_unused, s_sh):
    c = lax.axis_index("c")
    t = lax.axis_index("s")
    base = pl.multiple_of(c * _HALF, 8)
    pbase = pl.multiple_of(c * _EP, 8)
    dummy = _HALF + t
    off = pl.multiple_of(t * _SSTRIP, 32)

    z16f = jnp.zeros((16,), jnp.float32)
    for j in range(_K // 16):
        p4_v[pl.ds(j * 16, 16)] = z16f
    for k in range(_SSTRIP // _K):
        pltpu.sync_copy(p4_v.at[pl.ds(0, _K)],
                        s_sh.at[pl.ds(off + k * _K, _K)])
    pltpu.sync_copy(p4_v.at[pl.ds(0, _SSTRIP % _K)],
                    s_sh.at[pl.ds(off + (_SSTRIP // _K) * _K, _SSTRIP % _K)])

    pltpu.sync_copy(hs_hbm, hs_v)
    pltpu.sync_copy(hd_hbm.at[pl.ds(base, _HALF)], hd_v.at[pl.ds(0, _HALF)])
    plsc.subcore_barrier()

    ssems = (ssemA, ssemB)

    def sup(eoff, nsub, is_self, tail_sz=_K):
        szb = (nsub - 1) * _K + tail_sz
        if not is_self:
            pltpu.sync_copy(src_hbm.at[pl.ds(eoff, szb)],
                            src4_v.at[pl.ds(0, szb)])
            pltpu.sync_copy(dst_hbm.at[pl.ds(eoff, szb)],
                            dst4_v.at[pl.ds(0, szb)])
        pend = [None, None]
        for sub in range(nsub):
            sz = tail_sz if sub == nsub - 1 else _K

            def grp(j, carry):
                sl = pl.ds(j * 16, 16)
                if is_self:
                    s16 = eoff + sub * _K + j * 16 + lax.iota(jnp.int32, 16)
                    d16 = s16
                else:
                    s16 = src4_v[pl.ds(sub * _K + j * 16, 16)]
                    d16 = dst4_v[pl.ds(sub * _K + j * 16, 16)]
                hsg = plsc.load_gather(hs_v, [s16])
                dl16 = jnp.clip(d16 - base, 0, _HALF - 1)
                hdg = plsc.load_gather(hd_v, [dl16])
                l = hsg + hdg
                p16 = jnp.exp(jnp.where(l >= 0, l, 0.2 * l))
                owned = (d16 >= base) & (d16 < base + _HALF)
                dloc4_v[sub, sl] = jnp.where(owned, dl16, dummy)
                p4_v[pl.ds(sub * _K + j * 16, 16)] = p16
                return carry

            lax.fori_loop(0, sz // 16, grp, 0)
            if sz < _K:
                dummy16 = jnp.full((16,), dummy, jnp.int32)
                for j in range(sz // 16, _K // 16):
                    dloc4_v[sub, pl.ds(j * 16, 16)] = dummy16
            if pend[sub % 2] is not None:
                pend[sub % 2].wait()
            pend[sub % 2] = pltpu.async_copy(
                p4_v.at[pl.ds(sub * _K, _K)], s_sh.at[dloc4_v.at[sub]],
                ssems[sub % 2], add=True)
        for d in pend:
            if d is not None:
                d.wait()
        poff = pbase + (E + eoff if is_self else eoff)
        pltpu.sync_copy(p4_v.at[pl.ds(0, szb)], p_out.at[pl.ds(poff, szb)])

    _SKB = _SK * _K

    def eround(k, carry):
        sup((k * 16 + t) * _SKB, _SK, False)
        return carry

    lax.fori_loop(0, _NESUP // 16, eround, 0)

    def sround(k, carry):
        sup((k * 16 + t) * _SKB, _SK, True)
        return carry

    lax.fori_loop(0, _NSSUP // 16, sround, 0)

    if _NESUP % 16:
        @pl.when(t < _NESUP % 16)
        def _():
            sup(((_NESUP // 16) * 16 + t) * _SKB, _SK, False)

    if _NSSUP % 16:
        @pl.when(t < _NSSUP % 16)
        def _():
            sup(((_NSSUP // 16) * 16 + t) * _SKB, _SK, True)

    @pl.when(t == 13)
    def _():
        sup(_NESUP * _SKB, 1, False)

    @pl.when(t == 14)
    def _():
        sup(_NESUP * _SKB + _K, 1, False)

    @pl.when(t == 15)
    def _():
        sup(_NSSUP * _SKB, _NSCHUNK - _NSSUP * _SK + 1, True,
            tail_sz=_SELF_TAIL)

    plsc.subcore_barrier()
    for k in range(_SSTRIP // _K):
        pltpu.sync_copy(s_sh.at[pl.ds(off + k * _K, _K)],
                        p4_v.at[pl.ds(0, _K)])
        pltpu.sync_copy(p4_v.at[pl.ds(0, _K)],
                        s_out.at[pl.ds(c * _NP + off + k * _K, _K)])
    pltpu.sync_copy(s_sh.at[pl.ds(off + (_SSTRIP // _K) * _K, _SSTRIP % _K)],
                    p4_v.at[pl.ds(0, _SSTRIP % _K)])
    pltpu.sync_copy(p4_v.at[pl.ds(0, _SSTRIP % _K)],
                    s_out.at[pl.ds(c * _NP + off + (_SSTRIP // _K) * _K,
                                   _SSTRIP % _K)])


_attn_sc = functools.partial(
    pl.kernel,
    out_type=[jax.ShapeDtypeStruct((2 * _EP,), jnp.float32),
              jax.ShapeDtypeStruct((2 * _NP,), jnp.float32)],
    mesh=plsc.VectorSubcoreMesh(core_axis_name="c", subcore_axis_name="s"),
    compiler_params=pltpu.CompilerParams(needs_layout_passes=False,
                                         use_tc_tiling_on_sc=False),
    scratch_types=[
        pltpu.VMEM((N,), jnp.float32),
        pltpu.VMEM((_HALF + 24,), jnp.float32),
        pltpu.VMEM((_SK * _K,), jnp.int32),
        pltpu.VMEM((_SK * _K,), jnp.int32),
        pltpu.VMEM((_SK, _K), jnp.int32),
        pltpu.VMEM((_SK * _K,), jnp.float32),
        pltpu.SemaphoreType.DMA,
        pltpu.SemaphoreType.DMA,
        pltpu.SemaphoreType.DMA,
        pltpu.VMEM_SHARED((_NP,), jnp.float32),
    ],
)(_attn_body)


def _rows_body(h_hbm, src_hbm, dst_hbm, p_hbm,
               acc_out,
               src4_v, dst4_v, p4_v, dloc4_v, rows0_v, rows1_v,
               gsem0, gsem1, ssem0, ssem1, acc_sh):
    c = lax.axis_index("c")
    t = lax.axis_index("s")
    base = pl.multiple_of(c * _HALF, 8)
    pbase = pl.multiple_of(c * _EP, 8)
    dummy = _HALF + t
    off = pl.multiple_of(t * _STRIP, 32)

    z16f = jnp.zeros((16,), jnp.float32)

    def zrow(r, carry):
        for q in range(H // 16):
            rows0_v[r, pl.ds(q * 16, 16)] = z16f
        return carry

    lax.fori_loop(0, _K, zrow, 0)
    for k in range(12):
        pltpu.sync_copy(rows0_v, acc_sh.at[pl.ds(off + k * _K, _K)])
    pltpu.sync_copy(rows0_v.at[pl.ds(0, 32)],
                    acc_sh.at[pl.ds(off + 12 * _K, 32)])
    plsc.subcore_barrier()

    bufs = (rows0_v, rows1_v)
    gsems = (gsem0, gsem1)
    ssems = (ssem0, ssem1)

    def scale_sub(sub, buf, noff_or_none):
        def grp(j, carry):
            sl = pl.ds(j * 16, 16)
            if noff_or_none is None:
                d16 = dst4_v[pl.ds(sub * _K + j * 16, 16)]
            else:
                d16 = noff_or_none + sub * _K + j * 16 + lax.iota(jnp.int32, 16)
            owned = (d16 >= base) & (d16 < base + _HALF)
            dloc4_v[sub, sl] = jnp.where(owned, d16 - base, dummy)
            p16 = p4_v[pl.ds(sub * _K + j * 16, 16)]
            for e in range(16):
                pe = _splat(p16, e)
                row = j * 16 + e
                for q in range(H // 16):
                    cs = pl.ds(q * 16, 16)
                    buf[row, cs] = buf[row, cs] * pe
            return carry

        lax.fori_loop(0, _K // 16, grp, 0)

    def sup(eoff, nsub, is_self, tail_sz=_K):
        szb = (nsub - 1) * _K + tail_sz
        if is_self:
            pltpu.sync_copy(p_hbm.at[pl.ds(pbase + E + eoff, szb)],
                            p4_v.at[pl.ds(0, szb)])
        else:
            pltpu.sync_copy(src_hbm.at[pl.ds(eoff, szb)],
                            src4_v.at[pl.ds(0, szb)])
            pltpu.sync_copy(dst_hbm.at[pl.ds(eoff, szb)],
                            dst4_v.at[pl.ds(0, szb)])
            pltpu.sync_copy(p_hbm.at[pl.ds(pbase + eoff, szb)],
                            p4_v.at[pl.ds(0, szb)])

        def start_gather(sub):
            b = bufs[sub % 2]
            g = gsems[sub % 2]
            sz = tail_sz if sub == nsub - 1 else _K
            if is_self:
                return pltpu.async_copy(
                    h_hbm.at[pl.ds(eoff + sub * _K, sz)],
                    b.at[pl.ds(0, sz)], g)
            return pltpu.async_copy(
                h_hbm.at[src4_v.at[pl.ds(sub * _K, _K)]], b, gsems[sub % 2])

        pend_g = start_gather(0)
        pend_s = [None, None]
        for sub in range(nsub):
            buf = bufs[sub % 2]
            pend_g.wait()
            if sub + 1 < nsub:
                if pend_s[(sub + 1) % 2] is not None:
                    pend_s[(sub + 1) % 2].wait()
                    pend_s[(sub + 1) % 2] = None
                pend_g = start_gather(sub + 1)
            if is_self and sub == nsub - 1 and tail_sz < _K:
                dummy16 = jnp.full((16,), dummy, jnp.int32)
                for j in range(tail_sz // 16, _K // 16):
                    dloc4_v[sub, pl.ds(j * 16, 16)] = dummy16
                ngrp = tail_sz // 16
                def tgrp(j, carry):
                    sl = pl.ds(j * 16, 16)
                    d16 = eoff + sub * _K + j * 16 + lax.iota(jnp.int32, 16)
                    owned = (d16 >= base) & (d16 < base + _HALF)
                    dloc4_v[sub, sl] = jnp.where(owned, d16 - base, dummy)
                    p16 = p4_v[pl.ds(sub * _K + j * 16, 16)]
                    for e in range(16):
                        pe = _splat(p16, e)
                        row = j * 16 + e
                        for q in range(H // 16):
                            cs = pl.ds(q * 16, 16)
                            buf[row, cs] = buf[row, cs] * pe
                    return carry
                lax.fori_loop(0, ngrp, tgrp, 0)
            else:
                scale_sub(sub, buf, eoff if is_self else None)
            if pend_s[sub % 2] is not None:
                pend_s[sub % 2].wait()
            pend_s[sub % 2] = pltpu.async_copy(
                buf, acc_sh.at[dloc4_v.at[sub]], ssems[sub % 2], add=True)
        for d in pend_s:
            if d is not None:
                d.wait()

    _SKB = _SK * _K

    def eround(k, carry):
        sup((k * 16 + t) * _SKB, _SK, False)
        return carry

    lax.fori_loop(0, _NESUP // 16, eround, 0)

    def sround(k, carry):
        sup((k * 16 + t) * _SKB, _SK, True)
        return carry

    lax.fori_loop(0, _NSSUP // 16, sround, 0)

    if _NESUP % 16:
        @pl.when(t < _NESUP % 16)
        def _():
            sup(((_NESUP // 16) * 16 + t) * _SKB, _SK, False)

    if _NSSUP % 16:
        @pl.when(t < _NSSUP % 16)
        def _():
            sup(((_NSSUP // 16) * 16 + t) * _SKB, _SK, True)

    @pl.when(t == 13)
    def _():
        sup(_NESUP * _SKB, 1, False)

    @pl.when(t == 14)
    def _():
        sup(_NESUP * _SKB + _K, 1, False)

    @pl.when(t == 15)
    def _():
        sup(_NSSUP * _SKB, _NSCHUNK - _NSSUP * _SK + 1, True,
            tail_sz=_SELF_TAIL)

    plsc.subcore_barrier()
    for k in range(12):
        pltpu.sync_copy(acc_sh.at[pl.ds(off + k * _K, _K)], rows0_v)
        pltpu.sync_copy(rows0_v, acc_out.at[c, pl.ds(off + k * _K, _K)])
    pltpu.sync_copy(acc_sh.at[pl.ds(off + 12 * _K, 32)],
                    rows0_v.at[pl.ds(0, 32)])
    pltpu.sync_copy(rows0_v.at[pl.ds(0, 32)],
                    acc_out.at[c, pl.ds(off + 12 * _K, 32)])


_rows_sc = functools.partial(
    pl.kernel,
    out_type=[jax.ShapeDtypeStruct((2, _NP, H), jnp.float32)],
    mesh=plsc.VectorSubcoreMesh(core_axis_name="c", subcore_axis_name="s"),
    compiler_params=pltpu.CompilerParams(needs_layout_passes=False,
                                         use_tc_tiling_on_sc=False),
    scratch_types=[
        pltpu.VMEM((_SK * _K,), jnp.int32),
        pltpu.VMEM((_SK * _K,), jnp.int32),
        pltpu.VMEM((_SK * _K,), jnp.float32),
        pltpu.VMEM((_SK, _K), jnp.int32),
        pltpu.VMEM((_K, H), jnp.float32),
        pltpu.VMEM((_K, H), jnp.float32),
        pltpu.SemaphoreType.DMA,
        pltpu.SemaphoreType.DMA,
        pltpu.SemaphoreType.DMA,
        pltpu.SemaphoreType.DMA,
        pltpu.VMEM_SHARED((_NP, H), jnp.float32),
    ],
)(_rows_body)


def _edge_pass(h, hs, hd, src, dst):
    p_all, s_p = _attn_sc(hs.reshape(N), hd.reshape(N), src, dst)
    acc_p, = _rows_sc(h, src, dst, p_all)
    acc = jnp.concatenate([acc_p[0, :_HALF], acc_p[1, :_HALF]])
    s = jnp.concatenate([s_p[:_HALF], s_p[_NP:_NP + _HALF]])
    return acc, s.reshape(N, 1)


def kernel(x, edge_index, edge_attr, global_features, batch,
           W1, as1, ad1, b1, g1, bb1,
           W2, as2, ad2, b2, g2, bb2,
           W3, as3, ad3, b3, g3, bb3):
    src = edge_index[0]
    dst = edge_index[1]

    h, hs, hd = _mm1(x, W1, as1, ad1)
    acc, s = _edge_pass(h, hs, hd, src, dst)

    h, hs, hd = _lnmm(acc, s, b1, g1, bb1, W2, as2, ad2)
    acc, s = _edge_pass(h, hs, hd, src, dst)

    h, hs, hd = _lnmm(acc, s, b2, g2, bb2, W3, as3, ad3)
    acc, s = _edge_pass(h, hs, hd, src, dst)

    return _lnpool(acc, s, b3, g3, bb3, batch)
